# baseline (speedup 1.0000x reference)
import functools

import jax
import jax.numpy as jnp
from jax import lax
from jax.experimental import pallas as pl
from jax.experimental.pallas import tpu as pltpu
from jax.experimental.pallas import tpu_sc as plsc

BATCH = 16384
SEQ = 200
EMBED_DIM = 32
BB = 128
NBB = BATCH // BB
DB = EMBED_DIM // 8


@functools.cache
def _build():
    info = plsc.get_sparse_core_info()
    nc, ns = info.num_cores, info.num_subcores
    nw = nc * ns
    jb = NBB // nw

    mesh = plsc.VectorSubcoreMesh(core_axis_name="c", subcore_axis_name="s")

    @functools.partial(
        pl.kernel,
        mesh=mesh,
        compiler_params=pltpu.CompilerParams(
            use_tc_tiling_on_sc=False, needs_layout_passes=False
        ),
        out_type=jax.ShapeDtypeStruct((SEQ, DB, NBB, 8, 128), jnp.float32),
        scratch_types=[
            pltpu.VMEM((2, jb, BB), jnp.int32),
            pltpu.VMEM((2, jb, BB, EMBED_DIM), jnp.float32),
            pltpu.VMEM((2, jb, DB, 8, 128), jnp.float32),
            pltpu.SemaphoreType.DMA,
            pltpu.SemaphoreType.DMA,
        ],
    )
    def k(idx_hbm, table_hbm, out_hbm, idx_v, rows_v, tbuf, gsem, wsem):
        wid = lax.axis_index("s") * nc + lax.axis_index("c")
        wb = wid * jb

        def fire_gathers(s, bank):
            pltpu.sync_copy(idx_hbm.at[s, pl.ds(wb, jb)], idx_v.at[bank])
            for j in range(jb):
                pltpu.async_copy(
                    table_hbm.at[idx_v.at[bank, j]],
                    rows_v.at[bank, j],
                    gsem,
                )

        def drain_gathers(bank):
            for j in range(jb):
                pltpu.make_async_copy(
                    table_hbm.at[idx_v.at[bank, j]],
                    rows_v.at[bank, j],
                    gsem,
                ).wait()

        def drain_writes(s, bank):
            for j in range(jb):
                pltpu.make_async_copy(
                    tbuf.at[bank, j],
                    out_hbm.at[s, :, wb + j],
                    wsem,
                ).wait()

        biota = [lax.iota(jnp.int32, 16) + (jj * 16) for jj in range(8)]

        def transpose_block(bank, j):
            bidx = jnp.full((16,), bank, jnp.int32)
            jidx = jnp.full((16,), j, jnp.int32)
            for d in range(EMBED_DIM):
                didx = jnp.full((16,), d, jnp.int32)
                for jj in range(8):
                    v = plsc.load_gather(rows_v, [bidx, jidx, biota[jj], didx])
                    tbuf[bank, j, d // 8, d % 8, pl.ds(jj * 16, 16)] = v

        fire_gathers(0, 0)

        def half(s, t, bank):
            @pl.when(t >= 1)
            def _():
                drain_writes(s - 2, bank)

            drain_gathers(bank)

            @pl.when(s < SEQ - 1)
            def _():
                fire_gathers(s + 1, 1 - bank)

            for j in range(jb):
                transpose_block(bank, j)
                pltpu.async_copy(
                    tbuf.at[bank, j], out_hbm.at[s, :, wb + j], wsem
                )

        def body(t, carry):
            s0 = t * 2
            half(s0, t, 0)
            half(s0 + 1, t, 1)
            return carry

        lax.fori_loop(0, SEQ // 2, body, 0)
        drain_writes(SEQ - 2, 0)
        drain_writes(SEQ - 1, 1)

    return k


def kernel(phoneme_ids, embeddings_weight):
    ids3 = phoneme_ids.astype(jnp.int32).T.reshape(SEQ, NBB, BB)
    out5 = _build()(ids3, embeddings_weight)
    return out5.transpose(2, 4, 0, 1, 3).reshape(BATCH, SEQ, EMBED_DIM)


# speedup vs baseline: 1.3428x; 1.3428x over previous
import functools

import jax
import jax.numpy as jnp
from jax import lax
from jax.experimental import pallas as pl
from jax.experimental.pallas import tpu as pltpu
from jax.experimental.pallas import tpu_sc as plsc

BATCH = 16384
SEQ = 200
EMBED_DIM = 32
BB = 128
NBB = BATCH // BB
DB = EMBED_DIM // 8


@functools.cache
def _build():
    info = plsc.get_sparse_core_info()
    nc, ns = info.num_cores, info.num_subcores
    nw = nc * ns
    jb = NBB // nw

    mesh = plsc.VectorSubcoreMesh(core_axis_name="c", subcore_axis_name="s")

    @functools.partial(
        pl.kernel,
        mesh=mesh,
        compiler_params=pltpu.CompilerParams(
            use_tc_tiling_on_sc=False, needs_layout_passes=False
        ),
        out_type=jax.ShapeDtypeStruct((SEQ, DB, NBB, 1024), jnp.float32),
        scratch_types=[
            pltpu.VMEM((2, jb, BB), jnp.int32),
            pltpu.VMEM((2, jb, BB, EMBED_DIM), jnp.float32),
            pltpu.VMEM((2, jb, DB, 1024), jnp.float32),
            pltpu.SemaphoreType.DMA,
            pltpu.SemaphoreType.DMA,
        ],
    )
    def k(idx_hbm, table_hbm, out_hbm, idx_v, rows_v, tbuf, gsem, wsem):
        wid = lax.axis_index("s") * nc + lax.axis_index("c")
        wb = wid * jb

        def fire_gathers(s, bank):
            pltpu.sync_copy(idx_hbm.at[s, pl.ds(wb, jb)], idx_v.at[bank])
            for j in range(jb):
                pltpu.async_copy(
                    table_hbm.at[idx_v.at[bank, j]],
                    rows_v.at[bank, j],
                    gsem,
                )

        def drain_gathers(bank):
            for j in range(jb):
                pltpu.make_async_copy(
                    table_hbm.at[idx_v.at[bank, j]],
                    rows_v.at[bank, j],
                    gsem,
                ).wait()

        def drain_writes(s, bank):
            for j in range(jb):
                pltpu.make_async_copy(
                    tbuf.at[bank, j],
                    out_hbm.at[s, :, wb + j],
                    wsem,
                ).wait()

        biota = [lax.iota(jnp.int32, 16) + (jj * 16) for jj in range(8)]

        def transpose_block(bank, j):
            rows2 = rows_v.at[bank, j]

            @plsc.parallel_loop(0, EMBED_DIM, unroll=4)
            def _(d):
                dblk = d // 8
                base = (d % 8) * 128
                didx = jnp.full((16,), 0, jnp.int32) + d
                vs = [
                    plsc.load_gather(rows2, [biota[jj], didx])
                    for jj in range(8)
                ]
                for jj in range(8):
                    tbuf[bank, j, dblk, pl.ds(base + jj * 16, 16)] = vs[jj]

        fire_gathers(0, 0)

        def half(s, t, bank):
            @pl.when(t >= 1)
            def _():
                drain_writes(s - 2, bank)

            drain_gathers(bank)

            @pl.when(s < SEQ - 1)
            def _():
                fire_gathers(s + 1, 1 - bank)

            for j in range(jb):
                transpose_block(bank, j)
                pltpu.async_copy(
                    tbuf.at[bank, j], out_hbm.at[s, :, wb + j], wsem
                )

        def body(t, carry):
            s0 = t * 2
            half(s0, t, 0)
            half(s0 + 1, t, 1)
            return carry

        lax.fori_loop(0, SEQ // 2, body, 0)
        drain_writes(SEQ - 2, 0)
        drain_writes(SEQ - 1, 1)

    return k


def kernel(phoneme_ids, embeddings_weight):
    ids3 = phoneme_ids.astype(jnp.int32).T.reshape(SEQ, NBB, BB)
    out4 = _build()(ids3, embeddings_weight)
    out5 = out4.reshape(SEQ, DB, NBB, 8, 128)
    return out5.transpose(2, 4, 0, 1, 3).reshape(BATCH, SEQ, EMBED_DIM)


# speedup vs baseline: 1.4207x; 1.0581x over previous
import functools

import jax
import jax.numpy as jnp
from jax import lax
from jax.experimental import pallas as pl
from jax.experimental.pallas import tpu as pltpu
from jax.experimental.pallas import tpu_sc as plsc

BATCH = 16384
SEQ = 200
EMBED_DIM = 32
BB = 128
NBB = BATCH // BB
DB = EMBED_DIM // 8


@functools.cache
def _build():
    info = plsc.get_sparse_core_info()
    nc, ns = info.num_cores, info.num_subcores
    nw = nc * ns
    jb = NBB // nw

    mesh = plsc.VectorSubcoreMesh(core_axis_name="c", subcore_axis_name="s")

    @functools.partial(
        pl.kernel,
        mesh=mesh,
        compiler_params=pltpu.CompilerParams(
            use_tc_tiling_on_sc=False, needs_layout_passes=False
        ),
        out_type=jax.ShapeDtypeStruct((SEQ, DB, NBB, 1024), jnp.float32),
        scratch_types=[
            pltpu.VMEM((2, jb, BB), jnp.int32),
            pltpu.VMEM((2, jb, BB, EMBED_DIM), jnp.float32),
            pltpu.VMEM((2, jb, DB, 1024), jnp.float32),
            pltpu.SemaphoreType.DMA,
            pltpu.SemaphoreType.DMA,
            pltpu.SemaphoreType.DMA,
        ],
    )
    def k(idx_hbm, table_hbm, out_hbm, idx_v, rows_v, tbuf, gsem, wsem, isem):
        wid = lax.axis_index("s") * nc + lax.axis_index("c")
        wb = wid * jb

        def fire_idx(s, bank):
            pltpu.async_copy(
                idx_hbm.at[s, pl.ds(wb, jb)], idx_v.at[bank], isem
            )

        def drain_idx(s, bank):
            pltpu.make_async_copy(
                idx_hbm.at[s, pl.ds(wb, jb)], idx_v.at[bank], isem
            ).wait()

        def fire_gathers(s, bank):
            for j in range(jb):
                pltpu.async_copy(
                    table_hbm.at[idx_v.at[bank, j]],
                    rows_v.at[bank, j],
                    gsem,
                )

        def drain_gathers(bank):
            for j in range(jb):
                pltpu.make_async_copy(
                    table_hbm.at[idx_v.at[bank, j]],
                    rows_v.at[bank, j],
                    gsem,
                ).wait()

        def drain_writes(s, bank):
            for j in range(jb):
                pltpu.make_async_copy(
                    tbuf.at[bank, j],
                    out_hbm.at[s, :, wb + j],
                    wsem,
                ).wait()

        biota = [lax.iota(jnp.int32, 16) + (jj * 16) for jj in range(8)]

        def transpose_block(bank, j):
            rows2 = rows_v.at[bank, j]

            @plsc.parallel_loop(0, EMBED_DIM, unroll=4)
            def _(d):
                dblk = d // 8
                base = (d % 8) * 128
                didx = jnp.full((16,), 0, jnp.int32) + d
                vs = [
                    plsc.load_gather(rows2, [biota[jj], didx])
                    for jj in range(8)
                ]
                for jj in range(8):
                    tbuf[bank, j, dblk, pl.ds(base + jj * 16, 16)] = vs[jj]

        pltpu.sync_copy(idx_hbm.at[0, pl.ds(wb, jb)], idx_v.at[0])
        fire_gathers(0, 0)
        fire_idx(1, 1)

        def half(s, t, bank):
            @pl.when(t >= 1)
            def _():
                drain_writes(s - 2, bank)

            drain_gathers(bank)

            @pl.when(s < SEQ - 1)
            def _():
                drain_idx(s + 1, 1 - bank)
                fire_gathers(s + 1, 1 - bank)

            @pl.when(s < SEQ - 2)
            def _():
                fire_idx(s + 2, bank)

            for j in range(jb):
                transpose_block(bank, j)
                pltpu.async_copy(
                    tbuf.at[bank, j], out_hbm.at[s, :, wb + j], wsem
                )

        def body(t, carry):
            s0 = t * 2
            half(s0, t, 0)
            half(s0 + 1, t, 1)
            return carry

        lax.fori_loop(0, SEQ // 2, body, 0)
        drain_writes(SEQ - 2, 0)
        drain_writes(SEQ - 1, 1)

    return k


def kernel(phoneme_ids, embeddings_weight):
    ids3 = phoneme_ids.astype(jnp.int32).T.reshape(SEQ, NBB, BB)
    out4 = _build()(ids3, embeddings_weight)
    out5 = out4.reshape(SEQ, DB, NBB, 8, 128)
    return out5.transpose(2, 4, 0, 1, 3).reshape(BATCH, SEQ, EMBED_DIM)


# speedup vs baseline: 6.5517x; 4.6115x over previous
import functools

import jax
import jax.numpy as jnp
from jax import lax
from jax.experimental import pallas as pl
from jax.experimental.pallas import tpu as pltpu
from jax.experimental.pallas import tpu_sc as plsc

BATCH = 16384
SEQ = 200
EMBED_DIM = 32
BB = 128
NBB = BATCH // BB
DB = EMBED_DIM // 8
PW = 129


@functools.cache
def _build():
    info = plsc.get_sparse_core_info()
    nc, ns = info.num_cores, info.num_subcores
    nw = nc * ns
    jb = NBB // nw

    mesh = plsc.VectorSubcoreMesh(core_axis_name="c", subcore_axis_name="s")

    @functools.partial(
        pl.kernel,
        mesh=mesh,
        compiler_params=pltpu.CompilerParams(
            use_tc_tiling_on_sc=False, needs_layout_passes=False
        ),
        out_type=jax.ShapeDtypeStruct((SEQ, DB, NBB, 8, 128), jnp.float32),
        scratch_types=[
            pltpu.VMEM((2, jb, BB), jnp.int32),
            pltpu.VMEM((2, jb, BB, EMBED_DIM), jnp.float32),
            pltpu.VMEM((2, jb, DB, 8, PW), jnp.float32),
            pltpu.SemaphoreType.DMA,
            pltpu.SemaphoreType.DMA,
            pltpu.SemaphoreType.DMA,
        ],
    )
    def k(idx_hbm, table_hbm, out_hbm, idx_v, rows_v, tbuf, gsem, wsem, isem):
        wid = lax.axis_index("s") * nc + lax.axis_index("c")
        wb = wid * jb

        def fire_idx(s, bank):
            pltpu.async_copy(
                idx_hbm.at[s, pl.ds(wb, jb)], idx_v.at[bank], isem
            )

        def drain_idx(s, bank):
            pltpu.make_async_copy(
                idx_hbm.at[s, pl.ds(wb, jb)], idx_v.at[bank], isem
            ).wait()

        def fire_gathers(s, bank):
            for j in range(jb):
                pltpu.async_copy(
                    table_hbm.at[idx_v.at[bank, j]],
                    rows_v.at[bank, j],
                    gsem,
                )

        def drain_gathers(bank):
            for j in range(jb):
                pltpu.make_async_copy(
                    table_hbm.at[idx_v.at[bank, j]],
                    rows_v.at[bank, j],
                    gsem,
                ).wait()

        def out_copy(s, bank, j):
            return pltpu.make_async_copy(
                tbuf.at[bank, j, :, :, pl.ds(0, 128)],
                out_hbm.at[s, :, wb + j],
                wsem,
            )

        def drain_writes(s, bank):
            for j in range(jb):
                out_copy(s, bank, j).wait()

        iota = lax.iota(jnp.int32, 16)
        dblk_h = [(iota + 16 * h) // 8 for h in range(2)]
        din_h = [(iota + 16 * h) % 8 for h in range(2)]
        zero16 = jnp.zeros((16,), jnp.int32)

        def transpose_block(bank, j):
            tile = tbuf.at[bank, j]

            @plsc.parallel_loop(0, BB, unroll=8)
            def _(b):
                bfull = zero16 + b
                for h in range(2):
                    v = rows_v[bank, j, b, pl.ds(16 * h, 16)]
                    plsc.store_scatter(
                        tile, [dblk_h[h], din_h[h], bfull], v
                    )

        pltpu.sync_copy(idx_hbm.at[0, pl.ds(wb, jb)], idx_v.at[0])
        fire_gathers(0, 0)
        fire_idx(1, 1)

        def half(s, t, bank):
            @pl.when(t >= 1)
            def _():
                drain_writes(s - 2, bank)

            drain_gathers(bank)

            @pl.when(s < SEQ - 1)
            def _():
                drain_idx(s + 1, 1 - bank)
                fire_gathers(s + 1, 1 - bank)

            @pl.when(s < SEQ - 2)
            def _():
                fire_idx(s + 2, bank)

            for j in range(jb):
                transpose_block(bank, j)
                out_copy(s, bank, j).start()

        def body(t, carry):
            s0 = t * 2
            half(s0, t, 0)
            half(s0 + 1, t, 1)
            return carry

        lax.fori_loop(0, SEQ // 2, body, 0)
        drain_writes(SEQ - 2, 0)
        drain_writes(SEQ - 1, 1)

    return k


def kernel(phoneme_ids, embeddings_weight):
    ids3 = phoneme_ids.astype(jnp.int32).T.reshape(SEQ, NBB, BB)
    out5 = _build()(ids3, embeddings_weight)
    return out5.transpose(2, 4, 0, 1, 3).reshape(BATCH, SEQ, EMBED_DIM)


# speedup vs baseline: 7.2593x; 1.1080x over previous
import functools

import jax
import jax.numpy as jnp
from jax import lax
from jax.experimental import pallas as pl
from jax.experimental.pallas import tpu as pltpu
from jax.experimental.pallas import tpu_sc as plsc

BATCH = 16384
SEQ = 200
EMBED_DIM = 32
BB = 128
NBB = BATCH // BB
DB = EMBED_DIM // 8
PW = 129


@functools.cache
def _build():
    info = plsc.get_sparse_core_info()
    nc, ns = info.num_cores, info.num_subcores
    nw = nc * ns
    jb = NBB // nw

    mesh = plsc.VectorSubcoreMesh(core_axis_name="c", subcore_axis_name="s")

    @functools.partial(
        pl.kernel,
        mesh=mesh,
        compiler_params=pltpu.CompilerParams(
            use_tc_tiling_on_sc=False, needs_layout_passes=False
        ),
        out_type=jax.ShapeDtypeStruct((SEQ, DB, NBB, 8, 128), jnp.float32),
        scratch_types=[
            pltpu.VMEM((2, jb, BB), jnp.int32),
            pltpu.VMEM((2, jb, BB, EMBED_DIM), jnp.float32),
            pltpu.VMEM((2, jb, DB, 8, PW), jnp.float32),
            pltpu.SemaphoreType.DMA,
            pltpu.SemaphoreType.DMA,
            pltpu.SemaphoreType.DMA,
        ],
    )
    def k(idx_hbm, table_hbm, out_hbm, idx_v, rows_v, tbuf, gsem, wsem, isem):
        wid = lax.axis_index("s") * nc + lax.axis_index("c")
        wb = wid * jb

        def fire_idx(s, bank):
            pltpu.async_copy(
                idx_hbm.at[s, pl.ds(wb, jb)], idx_v.at[bank], isem
            )

        def drain_idx(s, bank):
            pltpu.make_async_copy(
                idx_hbm.at[s, pl.ds(wb, jb)], idx_v.at[bank], isem
            ).wait()

        def fire_gathers(s, bank):
            for j in range(jb):
                pltpu.async_copy(
                    table_hbm.at[idx_v.at[bank, j]],
                    rows_v.at[bank, j],
                    gsem,
                )

        def drain_gather(bank, j):
            pltpu.make_async_copy(
                table_hbm.at[idx_v.at[bank, j]],
                rows_v.at[bank, j],
                gsem,
            ).wait()

        def out_copy(s, bank, j):
            return pltpu.make_async_copy(
                tbuf.at[bank, j, :, :, pl.ds(0, 128)],
                out_hbm.at[s, :, wb + j],
                wsem,
            )

        def drain_writes(s, bank):
            for j in range(jb):
                out_copy(s, bank, j).wait()

        iota = lax.iota(jnp.int32, 16)
        dblk_h = [(iota + 16 * h) // 8 for h in range(2)]
        din_h = [(iota + 16 * h) % 8 for h in range(2)]
        zero16 = jnp.zeros((16,), jnp.int32)

        def transpose_block(bank, j):
            tile = tbuf.at[bank, j]

            @plsc.parallel_loop(0, BB, unroll=8)
            def _(b):
                bfull = zero16 + b
                for h in range(2):
                    v = rows_v[bank, j, b, pl.ds(16 * h, 16)]
                    plsc.store_scatter(
                        tile, [dblk_h[h], din_h[h], bfull], v
                    )

        pltpu.sync_copy(idx_hbm.at[0, pl.ds(wb, jb)], idx_v.at[0])
        fire_gathers(0, 0)
        fire_idx(1, 1)

        def half(s, t, bank):
            @pl.when(t >= 1)
            def _():
                drain_writes(s - 2, bank)

            drain_gather(bank, 0)

            @pl.when(s < SEQ - 1)
            def _():
                drain_idx(s + 1, 1 - bank)
                fire_gathers(s + 1, 1 - bank)

            @pl.when(s < SEQ - 2)
            def _():
                fire_idx(s + 2, bank)

            for j in range(jb):
                if j + 1 < jb:
                    drain_gather(bank, j + 1)
                transpose_block(bank, j)
                out_copy(s, bank, j).start()

        def body(t, carry):
            s0 = t * 2
            half(s0, t, 0)
            half(s0 + 1, t, 1)
            return carry

        lax.fori_loop(0, SEQ // 2, body, 0)
        drain_writes(SEQ - 2, 0)
        drain_writes(SEQ - 1, 1)

    return k


def kernel(phoneme_ids, embeddings_weight):
    ids3 = phoneme_ids.astype(jnp.int32).T.reshape(SEQ, NBB, BB)
    out5 = _build()(ids3, embeddings_weight)
    return out5.transpose(2, 4, 0, 1, 3).reshape(BATCH, SEQ, EMBED_DIM)
